# baseline (speedup 1.0000x reference)
import jax
import jax.numpy as jnp
from jax.experimental import pallas as pl
from jax.experimental.pallas import tpu as pltpu

_B, _N, _H, _O, _NBR, _HEADS = 1024, 64, 256, 128, 8, 3
_EPS = 1e-5
_Bb = 128
_NBLK = _B // _Bb


def _fused_kernel(obs_ref, w1_ref, b1_ref, g1_ref, be1_ref, w2_ref, b2_ref,
                  edges_ref, s24_ref, p192_ref, out_ref, a_s, c_s, q_s):
    nb2 = _NBR * _NBR
    i = pl.program_id(0)
    sel = jax.lax.rem(i, 2)
    rd = 1 - sel

    @pl.when(i == 0)
    def _build_coeffs():
        s_eff = (s24_ref[:, 0:_NBR] + s24_ref[:, _NBR:2 * _NBR]
                 + s24_ref[:, 2 * _NBR:3 * _NBR]) * (1.0 / _HEADS)
        p_mean = (p192_ref[:, 0:nb2] + p192_ref[:, nb2:2 * nb2]
                  + p192_ref[:, 2 * nb2:3 * nb2]) * (1.0 / _HEADS)
        ij = jax.lax.broadcasted_iota(jnp.int32, (_N, nb2), 1)
        tri = ((ij // _NBR) < (ij % _NBR)).astype(jnp.float32)
        p_eff = p_mean * tri
        l_i = jax.lax.broadcasted_iota(jnp.int32, (nb2, _NBR), 0)
        k_i = jax.lax.broadcasted_iota(jnp.int32, (nb2, _NBR), 1)
        rmask = ((l_i // _NBR) == k_i).astype(jnp.float32)
        cmask = ((l_i % _NBR) == k_i).astype(jnp.float32)
        rowsum = jnp.dot(p_eff, rmask, preferred_element_type=jnp.float32)
        colsum = jnp.dot(p_eff, cmask, preferred_element_type=jnp.float32)
        w = s_eff + 0.5 * (rowsum + colsum)
        m_iota = jax.lax.broadcasted_iota(jnp.int32, (_N, _N), 1)
        acc_a = (edges_ref[:, 0:1] == m_iota).astype(jnp.float32)
        for k in range(_NBR):
            hit = (edges_ref[:, k + 1:k + 2] == m_iota).astype(jnp.float32)
            acc_a = acc_a + w[:, k:k + 1] * hit
        a_s[...] = acc_a
        c_iota = jax.lax.broadcasted_iota(jnp.int32, (_N, _N * _N), 1)
        acc_c = jnp.zeros((_N, _N * _N), jnp.float32)
        for i2 in range(_NBR):
            for j2 in range(i2 + 1, _NBR):
                ei = edges_ref[:, i2 + 1:i2 + 2]
                ej = edges_ref[:, j2 + 1:j2 + 2]
                pa = jnp.minimum(ei, ej)
                pc = jnp.maximum(ei, ej)
                tt = pa // 8
                packed = ((8 * _N + 32) * tt - 32 * tt * tt
                          + (pa % 8) * (_N - 8 * tt) + (pc - 8 * tt))
                val = p_eff[:, i2 * _NBR + j2:i2 * _NBR + j2 + 1] * (-0.5)
                acc_c = acc_c + val * (packed == c_iota).astype(jnp.float32)
        c_s[...] = acc_c

    @pl.when(i < _NBLK)
    def _mmlp():
        ones_hb = jnp.ones((_H, 8), jnp.bfloat16)
        for n in range(_N):
            x = obs_ref[:, n, :]
            h = (jnp.dot(x, w1_ref[n], preferred_element_type=jnp.float32)
                 + b1_ref[n:n + 1, :])
            hb = h.astype(jnp.bfloat16)
            s1 = jnp.dot(hb, ones_hb, preferred_element_type=jnp.float32)
            s2 = jnp.dot(hb * hb, ones_hb, preferred_element_type=jnp.float32)
            mu = s1[:, 0:1] * (1.0 / _H)
            var = s2[:, 0:1] * (1.0 / _H) - mu * mu
            h = ((h - mu) * jax.lax.rsqrt(var + _EPS) * g1_ref[n:n + 1, :]
                 + be1_ref[n:n + 1, :])
            h = jnp.maximum(h, 0.0).astype(jnp.bfloat16)
            q = (jnp.dot(h, w2_ref[n], preferred_element_type=jnp.float32)
                 + b2_ref[n:n + 1, :])
            q_s[sel, n] = q

    @pl.when(i > 0)
    def _aggregate():
        tq = jnp.sum(q_s[rd], axis=2)
        acc = jnp.dot(a_s[...], tq, preferred_element_type=jnp.float32)
        cmat = c_s[...]
        for t in range(_N // 8):
            lo = 8 * t
            m = _N - lo
            off = (8 * _N + 32) * t - 32 * t * t
            qc = q_s[rd, lo:]
            ds = []
            for r in range(8):
                qa = q_s[rd, lo + r]
                ds.append(jnp.sum(jnp.abs(qc - qa[None]), axis=2))
            dd = jnp.concatenate(ds, axis=0)
            cslice = cmat[:, off:off + 8 * m]
            acc = acc + jnp.dot(cslice, dd, preferred_element_type=jnp.float32)
        out_ref[...] = acc * (1.0 / _O)


def kernel(observation, local_edges, W1, b1, g1, be1, W2, b2, singles, pairs):
    edges = local_edges[:, 0, :].astype(jnp.int32)
    s24 = singles.reshape(_N, _HEADS * _NBR)
    p192 = pairs.reshape(_N, _HEADS * _NBR * _NBR)
    obs_b = observation.astype(jnp.bfloat16)
    w1_b = W1.astype(jnp.bfloat16)
    w2_b = W2.astype(jnp.bfloat16)

    out_t = pl.pallas_call(
        _fused_kernel,
        grid=(_NBLK + 1,),
        in_specs=[
            pl.BlockSpec((_Bb, _N, _H), lambda i: (jnp.minimum(i, _NBLK - 1), 0, 0)),
            pl.BlockSpec((_N, _H, _H), lambda i: (0, 0, 0)),
            pl.BlockSpec((_N, _H), lambda i: (0, 0)),
            pl.BlockSpec((_N, _H), lambda i: (0, 0)),
            pl.BlockSpec((_N, _H), lambda i: (0, 0)),
            pl.BlockSpec((_N, _H, _O), lambda i: (0, 0, 0)),
            pl.BlockSpec((_N, _O), lambda i: (0, 0)),
            pl.BlockSpec((_N, _NBR + 1), lambda i: (0, 0)),
            pl.BlockSpec((_N, _HEADS * _NBR), lambda i: (0, 0)),
            pl.BlockSpec((_N, _HEADS * _NBR * _NBR), lambda i: (0, 0)),
        ],
        out_specs=pl.BlockSpec((_N, _Bb), lambda i: (0, jnp.maximum(i - 1, 0))),
        out_shape=jax.ShapeDtypeStruct((_N, _B), jnp.float32),
        scratch_shapes=[
            pltpu.VMEM((_N, _N), jnp.float32),
            pltpu.VMEM((_N, _N * _N), jnp.float32),
            pltpu.VMEM((2, _N, _Bb, _O), jnp.float32),
        ],
    )(obs_b, w1_b, b1, g1, be1, w2_b, b2, edges, s24, p192)
    return out_t.T


# speedup vs baseline: 1.2348x; 1.2348x over previous
import jax
import jax.numpy as jnp
from jax.experimental import pallas as pl
from jax.experimental.pallas import tpu as pltpu

_B, _N, _H, _O, _NBR, _HEADS = 1024, 64, 256, 128, 8, 3
_EPS = 1e-5
_Bb = 256
_NBLK = _B // _Bb


def _fused_kernel(obs_ref, w1_ref, b1_ref, g1_ref, be1_ref, w2_ref, b2_ref,
                  edges_ref, s24_ref, p192_ref, out_ref, a_s, c_s, q_s):
    nb2 = _NBR * _NBR
    i = pl.program_id(0)
    sel = jax.lax.rem(i, 2)
    rd = 1 - sel

    @pl.when(i == 0)
    def _build_coeffs():
        s_eff = (s24_ref[:, 0:_NBR] + s24_ref[:, _NBR:2 * _NBR]
                 + s24_ref[:, 2 * _NBR:3 * _NBR]) * (1.0 / _HEADS)
        p_mean = (p192_ref[:, 0:nb2] + p192_ref[:, nb2:2 * nb2]
                  + p192_ref[:, 2 * nb2:3 * nb2]) * (1.0 / _HEADS)
        ij = jax.lax.broadcasted_iota(jnp.int32, (_N, nb2), 1)
        tri = ((ij // _NBR) < (ij % _NBR)).astype(jnp.float32)
        p_eff = p_mean * tri
        l_i = jax.lax.broadcasted_iota(jnp.int32, (nb2, _NBR), 0)
        k_i = jax.lax.broadcasted_iota(jnp.int32, (nb2, _NBR), 1)
        rmask = ((l_i // _NBR) == k_i).astype(jnp.float32)
        cmask = ((l_i % _NBR) == k_i).astype(jnp.float32)
        rowsum = jnp.dot(p_eff, rmask, preferred_element_type=jnp.float32)
        colsum = jnp.dot(p_eff, cmask, preferred_element_type=jnp.float32)
        w = s_eff + 0.5 * (rowsum + colsum)
        m_iota = jax.lax.broadcasted_iota(jnp.int32, (_N, _N), 1)
        acc_a = (edges_ref[:, 0:1] == m_iota).astype(jnp.float32)
        for k in range(_NBR):
            hit = (edges_ref[:, k + 1:k + 2] == m_iota).astype(jnp.float32)
            acc_a = acc_a + w[:, k:k + 1] * hit
        a_s[...] = acc_a
        c_iota = jax.lax.broadcasted_iota(jnp.int32, (_N, _N * _N), 1)
        acc_c = jnp.zeros((_N, _N * _N), jnp.float32)
        for i2 in range(_NBR):
            for j2 in range(i2 + 1, _NBR):
                ei = edges_ref[:, i2 + 1:i2 + 2]
                ej = edges_ref[:, j2 + 1:j2 + 2]
                pa = jnp.minimum(ei, ej)
                pc = jnp.maximum(ei, ej)
                tt = pa // 8
                packed = ((8 * _N + 32) * tt - 32 * tt * tt
                          + (pa % 8) * (_N - 8 * tt) + (pc - 8 * tt))
                val = p_eff[:, i2 * _NBR + j2:i2 * _NBR + j2 + 1] * (-0.5)
                acc_c = acc_c + val * (packed == c_iota).astype(jnp.float32)
        c_s[...] = acc_c

    @pl.when(i < _NBLK)
    def _mmlp():
        for n in range(_N):
            x = obs_ref[:, n, :]
            h = (jnp.dot(x, w1_ref[n], preferred_element_type=jnp.float32)
                 + b1_ref[n:n + 1, :])
            mu = jnp.mean(h, axis=1, keepdims=True)
            hc = h - mu
            var = jnp.mean(hc * hc, axis=1, keepdims=True)
            h = (hc * jax.lax.rsqrt(var + _EPS) * g1_ref[n:n + 1, :]
                 + be1_ref[n:n + 1, :])
            h = jnp.maximum(h, 0.0).astype(jnp.bfloat16)
            q = (jnp.dot(h, w2_ref[n], preferred_element_type=jnp.float32)
                 + b2_ref[n:n + 1, :])
            q_s[sel, n] = q

    @pl.when(i > 0)
    def _aggregate():
        tq = jnp.sum(q_s[rd], axis=2)
        acc = jnp.dot(a_s[...], tq, preferred_element_type=jnp.float32)
        cmat = c_s[...]
        for t in range(_N // 8):
            lo = 8 * t
            m = _N - lo
            off = (8 * _N + 32) * t - 32 * t * t
            qc = q_s[rd, lo:]
            ds = []
            for r in range(8):
                qa = q_s[rd, lo + r]
                ds.append(jnp.sum(jnp.abs(qc - qa[None]), axis=2))
            dd = jnp.concatenate(ds, axis=0)
            cslice = cmat[:, off:off + 8 * m]
            acc = acc + jnp.dot(cslice, dd, preferred_element_type=jnp.float32)
        out_ref[...] = acc * (1.0 / _O)


def kernel(observation, local_edges, W1, b1, g1, be1, W2, b2, singles, pairs):
    edges = local_edges[:, 0, :].astype(jnp.int32)
    s24 = singles.reshape(_N, _HEADS * _NBR)
    p192 = pairs.reshape(_N, _HEADS * _NBR * _NBR)
    obs_b = observation.astype(jnp.bfloat16)
    w1_b = W1.astype(jnp.bfloat16)
    w2_b = W2.astype(jnp.bfloat16)

    out_t = pl.pallas_call(
        _fused_kernel,
        grid=(_NBLK + 1,),
        in_specs=[
            pl.BlockSpec((_Bb, _N, _H), lambda i: (jnp.minimum(i, _NBLK - 1), 0, 0)),
            pl.BlockSpec((_N, _H, _H), lambda i: (0, 0, 0)),
            pl.BlockSpec((_N, _H), lambda i: (0, 0)),
            pl.BlockSpec((_N, _H), lambda i: (0, 0)),
            pl.BlockSpec((_N, _H), lambda i: (0, 0)),
            pl.BlockSpec((_N, _H, _O), lambda i: (0, 0, 0)),
            pl.BlockSpec((_N, _O), lambda i: (0, 0)),
            pl.BlockSpec((_N, _NBR + 1), lambda i: (0, 0)),
            pl.BlockSpec((_N, _HEADS * _NBR), lambda i: (0, 0)),
            pl.BlockSpec((_N, _HEADS * _NBR * _NBR), lambda i: (0, 0)),
        ],
        out_specs=pl.BlockSpec((_N, _Bb), lambda i: (0, jnp.maximum(i - 1, 0))),
        out_shape=jax.ShapeDtypeStruct((_N, _B), jnp.float32),
        scratch_shapes=[
            pltpu.VMEM((_N, _N), jnp.float32),
            pltpu.VMEM((_N, _N * _N), jnp.float32),
            pltpu.VMEM((2, _N, _Bb, _O), jnp.float32),
        ],
    )(obs_b, w1_b, b1, g1, be1, w2_b, b2, edges, s24, p192)
    return out_t.T
